# Initial kernel scaffold; baseline (speedup 1.0000x reference)
import jax
import jax.numpy as jnp
from jax.experimental import pallas as pl


def kernel(x, pe_table):
    raise NotImplementedError("write your pallas kernel here")


# speedup vs baseline: 2.8191x; 2.8191x over previous
import jax
import jax.numpy as jnp
from jax.experimental import pallas as pl

B, T, D = 4, 8192, 1024
TB = 512


def _add_body(x_ref, pe_ref, out_ref):
    out_ref[...] = x_ref[...] + pe_ref[...][None]


def kernel(x, pe_table):
    return pl.pallas_call(
        _add_body,
        grid=(T // TB, B),
        in_specs=[
            pl.BlockSpec((1, TB, D), lambda t, b: (b, t, 0)),
            pl.BlockSpec((TB, D), lambda t, b: (t, 0)),
        ],
        out_specs=pl.BlockSpec((1, TB, D), lambda t, b: (b, t, 0)),
        out_shape=jax.ShapeDtypeStruct((B, T, D), jnp.float32),
    )(x, pe_table)
